# Initial kernel scaffold; baseline (speedup 1.0000x reference)
import jax
import jax.numpy as jnp
from jax.experimental import pallas as pl


def kernel(userIdx, itemIdx, uEmbd, iEmbd, lap_row, lap_col, lap_val, ga0_W1, ga0_b1, ga0_W2, ga0_b2, ga1_W1, ga1_b1, ga1_W2, ga1_b2, t1_W, t1_b, t2_W, t2_b, t3_W, t3_b):
    raise NotImplementedError("write your pallas kernel here")


# speedup vs baseline: 7.9081x; 7.9081x over previous
import functools

import jax
import jax.numpy as jnp
from jax import lax
from jax.experimental import pallas as pl
from jax.experimental.pallas import tpu as pltpu
from jax.experimental.pallas import tpu_sc as plsc

N_USERS = 5000
N_NODES = 10000
NC = 2
NS = 16
NW = NC * NS
LANES = 16
EDGE_B = 128


def _mesh():
    return plsc.VectorSubcoreMesh(core_axis_name="c", subcore_axis_name="s",
                                  num_cores=NC, num_subcores=NS)


def _sc_spmm(rows, cols, vals, Z):
    E = rows.shape[0]
    D = Z.shape[1]
    NB = E // EDGE_B
    TPW = -(-NB // NW)
    CH = 80
    NCH = N_NODES // CH
    CPT = -(-NCH // NS)

    @functools.partial(
        pl.kernel,
        out_type=jax.ShapeDtypeStruct((NC, N_NODES, D), jnp.float32),
        mesh=_mesh(),
        scratch_types=[
            pltpu.VMEM((EDGE_B,), jnp.int32),
            pltpu.VMEM((1, EDGE_B), jnp.int32),
            pltpu.VMEM((EDGE_B,), jnp.float32),
            pltpu.VMEM((EDGE_B, D), jnp.float32),
            pltpu.VMEM((EDGE_B, D), jnp.float32),
            pltpu.VMEM_SHARED((N_NODES, D), jnp.float32),
            pltpu.SemaphoreType.DMA,
        ],
    )
    def k(rows_hbm, cols_hbm, vals_hbm, z_hbm, out_hbm,
          cols_v, rows_v, vals_v, zbuf, sbuf, acc, sem):
        c = lax.axis_index("c")
        s = lax.axis_index("s")
        wid = c * NS + s

        zero16 = jnp.zeros((LANES,), jnp.float32)

        def zrow(i, _):
            for j in range(D // LANES):
                sbuf[i, pl.ds(j * LANES, LANES)] = zero16
            return 0

        lax.fori_loop(0, EDGE_B, zrow, 0)
        for i in range(CPT):
            ch = s + i * NS

            @pl.when(ch < NCH)
            def _():
                pltpu.sync_copy(sbuf.at[pl.ds(0, CH)],
                                acc.at[pl.ds(ch * CH, CH)])

        plsc.subcore_barrier()

        def batch(t, _):
            bidx = wid + t * NW

            @pl.when(bidx < NB)
            def _():
                base = bidx * EDGE_B
                pltpu.sync_copy(cols_hbm.at[pl.ds(base, EDGE_B)], cols_v)
                pltpu.sync_copy(rows_hbm.at[pl.ds(base, EDGE_B)], rows_v.at[0])
                pltpu.sync_copy(vals_hbm.at[pl.ds(base, EDGE_B)], vals_v)
                pltpu.async_copy(z_hbm.at[cols_v], zbuf, sem).wait()

                def edge_group(g, _):
                    vv = vals_v[pl.ds(g * LANES, LANES)]
                    for k in range(LANES):
                        e = g * LANES + k
                        val = vv[k]
                        for j in range(D // LANES):
                            sl = pl.ds(j * LANES, LANES)
                            sbuf[e, sl] = zbuf[e, sl] * val
                    return 0

                lax.fori_loop(0, EDGE_B // LANES, edge_group, 0)
                pltpu.sync_copy(sbuf, acc.at[rows_v.at[0]], add=True)

            return 0

        lax.fori_loop(0, TPW, batch, 0)
        plsc.subcore_barrier()

        for i in range(CPT):
            ch = s + i * NS

            @pl.when(ch < NCH)
            def _():
                r0 = ch * CH
                pltpu.sync_copy(acc.at[pl.ds(r0, CH)], zbuf.at[pl.ds(0, CH)])
                pltpu.sync_copy(zbuf.at[pl.ds(0, CH)],
                                out_hbm.at[c, pl.ds(r0, CH)])

    return k(rows, cols, vals, Z)


def _sc_gather(user_idx, item_idx2, feats0, feats1, p0, p1, a1b):
    B = user_idx.shape[0]
    RB = B // NW

    @functools.partial(
        pl.kernel,
        out_type=[
            jax.ShapeDtypeStruct((B, 256), jnp.float32),
            jax.ShapeDtypeStruct((B, 128), jnp.float32),
            jax.ShapeDtypeStruct((B, 64), jnp.float32),
            jax.ShapeDtypeStruct((B, 256), jnp.float32),
            jax.ShapeDtypeStruct((B, 128), jnp.float32),
            jax.ShapeDtypeStruct((B, 64), jnp.float32),
        ],
        mesh=_mesh(),
        scratch_types=[
            pltpu.VMEM((RB,), jnp.int32),
            pltpu.VMEM((RB, 256), jnp.float32),
            pltpu.VMEM((RB, 128), jnp.float32),
            pltpu.VMEM((RB, 128), jnp.float32),
            pltpu.VMEM((RB, 128), jnp.float32),
            pltpu.VMEM((RB, 128), jnp.float32),
            pltpu.VMEM((RB, 64), jnp.float32),
            pltpu.SemaphoreType.DMA,
        ],
    )
    def k(u_hbm, i_hbm, f0_hbm, f1_hbm, p0_hbm, p1_hbm, a1b_hbm,
          o0u, o1u, o2u, o0i, o1i, o2i,
          idx_v, g0, g1, ga, gb, gc, gsum, sem):
        c = lax.axis_index("c")
        s = lax.axis_index("s")
        wid = c * NS + s
        base = wid * RB

        for idx_hbm, o0, o1, o2 in ((u_hbm, o0u, o1u, o2u),
                                    (i_hbm, o0i, o1i, o2i)):
            pltpu.sync_copy(idx_hbm.at[pl.ds(base, RB)], idx_v)
            pltpu.async_copy(f0_hbm.at[idx_v], g0, sem).wait()
            pltpu.sync_copy(g0, o0.at[pl.ds(base, RB)])
            pltpu.async_copy(f1_hbm.at[idx_v], g1, sem).wait()
            pltpu.sync_copy(g1, o1.at[pl.ds(base, RB)])
            pltpu.async_copy(p0_hbm.at[idx_v], ga, sem).wait()
            pltpu.async_copy(p1_hbm.at[idx_v], gb, sem).wait()
            pltpu.async_copy(a1b_hbm.at[idx_v], gc, sem).wait()

            def addrow(i, _):
                for j in range(64 // LANES):
                    sl = pl.ds(j * LANES, LANES)
                    gsum[i, sl] = ga[i, sl] + gb[i, sl] + gc[i, sl]
                return 0

            lax.fori_loop(0, RB, addrow, 0)
            pltpu.sync_copy(gsum, o2.at[pl.ds(base, RB)])

    return k(user_idx, item_idx2, feats0, feats1, p0, p1, a1b)


def _tc_layer(X, w1t, w2t, bsum, rb):
    n, din = X.shape
    dout = w1t.shape[1]

    def body(x_ref, w1_ref, w2_ref, b_ref, z_ref, ab_ref):
        x = x_ref[...]
        a = jnp.dot(x, w1_ref[...], preferred_element_type=jnp.float32)
        b = jnp.dot(x * x, w2_ref[...], preferred_element_type=jnp.float32)
        z_ref[...] = a + b
        ab_ref[...] = a + b_ref[...]

    return pl.pallas_call(
        body,
        grid=(n // rb,),
        in_specs=[
            pl.BlockSpec((rb, din), lambda i: (i, 0)),
            pl.BlockSpec((din, dout), lambda i: (0, 0)),
            pl.BlockSpec((din, dout), lambda i: (0, 0)),
            pl.BlockSpec((1, dout), lambda i: (0, 0)),
        ],
        out_specs=[
            pl.BlockSpec((rb, dout), lambda i: (i, 0)),
            pl.BlockSpec((rb, dout), lambda i: (i, 0)),
        ],
        out_shape=[
            jax.ShapeDtypeStruct((n, dout), jnp.float32),
            jax.ShapeDtypeStruct((n, dout), jnp.float32),
        ],
    )(X, w1t, w2t, bsum)


def _tc_layer2_in(p0, p1, a0b, w1t, w2t, bsum, rb):
    n, din = a0b.shape
    dout = w1t.shape[1]

    def body(p0_ref, p1_ref, ab_ref, w1_ref, w2_ref, b_ref,
             f_ref, z_ref, a1b_ref):
        f = p0_ref[...] + p1_ref[...] + ab_ref[...]
        f_ref[...] = f
        a = jnp.dot(f, w1_ref[...], preferred_element_type=jnp.float32)
        b = jnp.dot(f * f, w2_ref[...], preferred_element_type=jnp.float32)
        z_ref[...] = a + b
        a1b_ref[...] = a + b_ref[...]

    return pl.pallas_call(
        body,
        grid=(n // rb,),
        in_specs=[
            pl.BlockSpec((rb, din), lambda i: (i, 0)),
            pl.BlockSpec((rb, din), lambda i: (i, 0)),
            pl.BlockSpec((rb, din), lambda i: (i, 0)),
            pl.BlockSpec((din, dout), lambda i: (0, 0)),
            pl.BlockSpec((din, dout), lambda i: (0, 0)),
            pl.BlockSpec((1, dout), lambda i: (0, 0)),
        ],
        out_specs=[
            pl.BlockSpec((rb, din), lambda i: (i, 0)),
            pl.BlockSpec((rb, dout), lambda i: (i, 0)),
            pl.BlockSpec((rb, dout), lambda i: (i, 0)),
        ],
        out_shape=[
            jax.ShapeDtypeStruct((n, din), jnp.float32),
            jax.ShapeDtypeStruct((n, dout), jnp.float32),
            jax.ShapeDtypeStruct((n, dout), jnp.float32),
        ],
    )(p0, p1, a0b, w1t, w2t, bsum)


def _tc_mlp(xs, w1s, t1b, t2t, t2b, t3, t3b, rb):
    n = xs[0].shape[0]
    dks = [x.shape[1] for x in xs]

    def body(*refs):
        x_refs = refs[0:6]
        w1_refs = refs[6:12]
        b1_ref, w2_ref, b2_ref, w3_ref, b3_ref, o_ref = refs[12:]
        h = b1_ref[...]
        for xr, wr in zip(x_refs, w1_refs):
            h = h + jnp.dot(xr[...], wr[...], preferred_element_type=jnp.float32)
        h = jax.nn.relu(h)
        h = jax.nn.relu(jnp.dot(h, w2_ref[...],
                                preferred_element_type=jnp.float32) + b2_ref[...])
        o_ref[...] = jnp.sum(h * w3_ref[...], axis=1, keepdims=True) + b3_ref[...]

    return pl.pallas_call(
        body,
        grid=(n // rb,),
        in_specs=[pl.BlockSpec((rb, dk), lambda i: (i, 0)) for dk in dks]
        + [pl.BlockSpec((dk, 64), lambda i: (0, 0)) for dk in dks]
        + [
            pl.BlockSpec((1, 64), lambda i: (0, 0)),
            pl.BlockSpec((64, 32), lambda i: (0, 0)),
            pl.BlockSpec((1, 32), lambda i: (0, 0)),
            pl.BlockSpec((1, 32), lambda i: (0, 0)),
            pl.BlockSpec((1, 1), lambda i: (0, 0)),
        ],
        out_specs=pl.BlockSpec((rb, 1), lambda i: (i, 0)),
        out_shape=jax.ShapeDtypeStruct((n, 1), jnp.float32),
    )(*xs, *w1s, t1b, t2t, t2b, t3, t3b)


def kernel(userIdx, itemIdx, uEmbd, iEmbd, lap_row, lap_col, lap_val,
           ga0_W1, ga0_b1, ga0_W2, ga0_b2, ga1_W1, ga1_b1, ga1_W2, ga1_b2,
           t1_W, t1_b, t2_W, t2_b, t3_W, t3_b):
    feats0 = jnp.concatenate([uEmbd, iEmbd], axis=0)

    z0, a0b = _tc_layer(feats0, ga0_W1.T, ga0_W2.T,
                        (ga0_b1 + ga0_b2)[None, :], 1000)
    part0 = _sc_spmm(lap_row, lap_col, lap_val, z0)
    pad = jnp.zeros((128, 64), jnp.float32)
    w1t1 = jnp.concatenate([ga1_W1.T, pad], axis=1)
    w2t1 = jnp.concatenate([ga1_W2.T, pad], axis=1)
    bsum1 = jnp.concatenate([ga1_b1 + ga1_b2, jnp.zeros((64,), jnp.float32)])
    feats1, z1, a1b = _tc_layer2_in(part0[0], part0[1], a0b,
                                    w1t1, w2t1, bsum1[None, :], 1000)
    part1 = _sc_spmm(lap_row, lap_col, lap_val, z1)

    g0u, g1u, g2u, g0i, g1i, g2i = _sc_gather(
        userIdx, itemIdx + N_USERS, feats0, feats1, part1[0], part1[1], a1b)

    t1t = t1_W.T
    w1s = (t1t[0:256], t1t[256:384], t1t[384:448],
           t1t[448:704], t1t[704:832], t1t[832:896])
    out = _tc_mlp((g0u, g1u, g2u, g0i, g1i, g2i), w1s,
                  t1_b[None, :], t2_W.T, t2_b[None, :],
                  t3_W, t3_b[None, :], 512)
    return out.reshape(-1)
